# baseline (speedup 1.0000x reference)
import jax
import jax.numpy as jnp
from jax import lax
from jax.experimental import pallas as pl
from jax.experimental.pallas import tpu as pltpu
from jax.experimental.pallas import tpu_sc as plsc

VOCAB = 1000000
D = 64
SEQ = 200
BATCH = 4096

NC = 2
NS = 16
NW = NC * NS
TOTAL_ROWS = BATCH * SEQ
ROWS_PER_W = TOTAL_ROWS // NW
CHUNK = SEQ
CHUNKS_PER_W = ROWS_PER_W // CHUNK


def _positional_encoding_table():
    pos = jnp.arange(SEQ, dtype=jnp.float32)[:, None]
    i = jnp.arange(0, D, 2, dtype=jnp.float32)
    div = jnp.exp(-jnp.log(10000.0) * i / D)
    ang = pos * div[None, :]
    pe = jnp.zeros((SEQ, D), dtype=jnp.float32)
    pe = pe.at[:, 0::2].set(jnp.sin(ang))
    pe = pe.at[:, 1::2].set(jnp.cos(ang))
    return pe


def _sc_body(table_hbm, idx_hbm, pe_hbm, out_hbm,
             idx_v, pe_v, buf_v, gsem, ssem):
    wid = lax.axis_index("s") * NC + lax.axis_index("c")
    base = wid * ROWS_PER_W

    pltpu.sync_copy(idx_hbm.at[pl.ds(base, ROWS_PER_W)], idx_v)
    pltpu.sync_copy(pe_hbm, pe_v)

    @pl.loop(0, CHUNKS_PER_W)
    def _chunk(c):
        start = c * CHUNK
        cp0 = pltpu.async_copy(
            table_hbm.at[idx_v.at[pl.ds(start, 128)]],
            buf_v.at[pl.ds(0, 128)], gsem)
        cp1 = pltpu.async_copy(
            table_hbm.at[idx_v.at[pl.ds(start + 128, CHUNK - 128)]],
            buf_v.at[pl.ds(128, CHUNK - 128)], gsem)
        cp0.wait()
        cp1.wait()

        @pl.loop(0, CHUNK, unroll=4)
        def _row(r):
            for j in range(D // 16):
                sl = pl.ds(j * 16, 16)
                plsc.addupdate(buf_v.at[r, sl], pe_v[r, sl])

        pltpu.async_copy(buf_v, out_hbm.at[pl.ds(base + start, CHUNK)],
                         ssem).wait()


@jax.jit
def _embed(x, token_emb, pe):
    xf = x.reshape(TOTAL_ROWS).astype(jnp.int32)
    mesh = plsc.VectorSubcoreMesh(core_axis_name="c", subcore_axis_name="s")
    out = pl.kernel(
        _sc_body,
        out_type=jax.ShapeDtypeStruct((TOTAL_ROWS, D), jnp.float32),
        mesh=mesh,
        compiler_params=pltpu.CompilerParams(use_tc_tiling_on_sc=False),
        scratch_types=[
            pltpu.VMEM((ROWS_PER_W,), jnp.int32),
            pltpu.VMEM((SEQ, D), jnp.float32),
            pltpu.VMEM((CHUNK, D), jnp.float32),
            pltpu.SemaphoreType.DMA,
            pltpu.SemaphoreType.DMA,
        ],
    )(token_emb, xf, pe)
    return out.reshape(BATCH, SEQ, D)


def kernel(x, token_emb):
    pe = _positional_encoding_table()
    return _embed(x, token_emb, pe)


# speedup vs baseline: 1.1228x; 1.1228x over previous
import jax
import jax.numpy as jnp
from jax import lax
from jax.experimental import pallas as pl
from jax.experimental.pallas import tpu as pltpu
from jax.experimental.pallas import tpu_sc as plsc

VOCAB = 1000000
D = 64
SEQ = 200
BATCH = 4096

NC = 2
NS = 16
NW = NC * NS
TOTAL_ROWS = BATCH * SEQ
ROWS_PER_W = TOTAL_ROWS // NW
CHUNK = SEQ
CHUNKS_PER_W = ROWS_PER_W // CHUNK
NBUF = 4
SPLIT = 128
REST = CHUNK - SPLIT


def _positional_encoding_table():
    pos = jnp.arange(SEQ, dtype=jnp.float32)[:, None]
    i = jnp.arange(0, D, 2, dtype=jnp.float32)
    div = jnp.exp(-jnp.log(10000.0) * i / D)
    ang = pos * div[None, :]
    pe = jnp.zeros((SEQ, D), dtype=jnp.float32)
    pe = pe.at[:, 0::2].set(jnp.sin(ang))
    pe = pe.at[:, 1::2].set(jnp.cos(ang))
    return pe


def _sc_body(table_hbm, idx_hbm, pe_hbm, out_hbm,
             idx_v, pe_v, bufs, gsems, ssems):
    wid = lax.axis_index("s") * NC + lax.axis_index("c")
    base = wid * ROWS_PER_W

    pltpu.sync_copy(idx_hbm.at[pl.ds(base, ROWS_PER_W)], idx_v)
    pltpu.sync_copy(pe_hbm, pe_v)

    def start_gather(c, b):
        start = c * CHUNK
        pltpu.async_copy(table_hbm.at[idx_v.at[pl.ds(start, SPLIT)]],
                         bufs.at[b, pl.ds(0, SPLIT)], gsems.at[b])
        pltpu.async_copy(table_hbm.at[idx_v.at[pl.ds(start + SPLIT, REST)]],
                         bufs.at[b, pl.ds(SPLIT, REST)], gsems.at[b])

    def wait_gather(b):
        pltpu.make_async_copy(table_hbm.at[idx_v.at[pl.ds(0, SPLIT)]],
                              bufs.at[b, pl.ds(0, SPLIT)], gsems.at[b]).wait()
        pltpu.make_async_copy(table_hbm.at[idx_v.at[pl.ds(0, REST)]],
                              bufs.at[b, pl.ds(SPLIT, REST)], gsems.at[b]).wait()

    def drain_store(b):
        pltpu.make_async_copy(bufs.at[b], out_hbm.at[pl.ds(0, CHUNK)],
                              ssems.at[b]).wait()

    for b in range(NBUF - 1):
        start_gather(b, b)

    @pl.loop(0, CHUNKS_PER_W // NBUF)
    def _grp(g):
        c0 = g * NBUF
        for b in range(NBUF):
            c = c0 + b
            bb = (b + NBUF - 1) % NBUF

            @pl.when(c + NBUF - 1 < CHUNKS_PER_W)
            def _():
                @pl.when(c >= 1)
                def _():
                    drain_store(bb)
                start_gather(c + NBUF - 1, bb)

            wait_gather(b)

            @pl.loop(0, CHUNK, unroll=8)
            def _row(r):
                for j in range(D // 16):
                    sl = pl.ds(j * 16, 16)
                    plsc.addupdate(bufs.at[b, r, sl], pe_v[r, sl])

            pltpu.async_copy(bufs.at[b],
                             out_hbm.at[pl.ds(base + c * CHUNK, CHUNK)],
                             ssems.at[b])

    for b in range(NBUF):
        drain_store(b)


@jax.jit
def _embed(x, token_emb, pe):
    xf = x.reshape(TOTAL_ROWS).astype(jnp.int32)
    mesh = plsc.VectorSubcoreMesh(core_axis_name="c", subcore_axis_name="s")
    out = pl.kernel(
        _sc_body,
        out_type=jax.ShapeDtypeStruct((TOTAL_ROWS, D), jnp.float32),
        mesh=mesh,
        compiler_params=pltpu.CompilerParams(use_tc_tiling_on_sc=False),
        scratch_types=[
            pltpu.VMEM((ROWS_PER_W,), jnp.int32),
            pltpu.VMEM((SEQ, D), jnp.float32),
            pltpu.VMEM((NBUF, CHUNK, D), jnp.float32),
            pltpu.SemaphoreType.DMA((NBUF,)),
            pltpu.SemaphoreType.DMA((NBUF,)),
        ],
    )(token_emb, xf, pe)
    return out.reshape(BATCH, SEQ, D)


def kernel(x, token_emb):
    pe = _positional_encoding_table()
    return _embed(x, token_emb, pe)
